# Initial kernel scaffold; baseline (speedup 1.0000x reference)
import jax
import jax.numpy as jnp
from jax.experimental import pallas as pl


def kernel(x, edge_index, edge_attr, batch, W1, b1, W2, b2, Wr, br):
    raise NotImplementedError("write your pallas kernel here")


# speedup vs baseline: 6.4679x; 6.4679x over previous
import functools

import jax
import jax.numpy as jnp
from jax import lax
from jax.experimental import pallas as pl
from jax.experimental.pallas import tpu as pltpu
from jax.experimental.pallas import tpu_sc as plsc

N = 10000
E = 320000
D_IN = 128
D_H = 256
D_OUT = 128
B = 32

NC = 2
NS = 16
NPAD = 10240
SLC = NPAD // NS
K = 80

_mesh = plsc.VectorSubcoreMesh(core_axis_name="c", subcore_axis_name="s")


@functools.partial(
    pl.kernel,
    out_type=jax.ShapeDtypeStruct((NC, NPAD), jnp.float32),
    mesh=_mesh,
    scratch_types=[
        pltpu.VMEM((K,), jnp.int32),
        pltpu.VMEM((K,), jnp.float32),
        pltpu.VMEM_SHARED((NPAD,), jnp.float32),
    ],
)
def _deg_kernel(col_hbm, w_hbm, z1_hbm, out_hbm, cidx, wbuf, sdeg):
    c = lax.axis_index("c")
    s = lax.axis_index("s")
    pltpu.sync_copy(z1_hbm, sdeg.at[pl.ds(s * SLC, SLC)])
    plsc.subcore_barrier()
    ept = E // (NC * NS)
    base = (s * NC + c) * ept

    @pl.loop(0, ept // K)
    def _chunk(i):
        b = base + i * K
        pltpu.sync_copy(col_hbm.at[pl.ds(b, K)], cidx)
        pltpu.sync_copy(w_hbm.at[pl.ds(b, K)], wbuf)
        pltpu.sync_copy(wbuf, sdeg.at[cidx], add=True)

    plsc.subcore_barrier()
    pltpu.sync_copy(sdeg.at[pl.ds(s * SLC, SLC)], out_hbm.at[c, pl.ds(s * SLC, SLC)])


@functools.partial(
    pl.kernel,
    out_type=jax.ShapeDtypeStruct((NC, NPAD, 128), jnp.float32),
    mesh=_mesh,
    scratch_types=[
        pltpu.VMEM((K,), jnp.int32),
        pltpu.VMEM((K,), jnp.int32),
        pltpu.VMEM((K,), jnp.int32),
        pltpu.VMEM((K,), jnp.float32),
        pltpu.VMEM((K, 128), jnp.float32),
        pltpu.VMEM_SHARED((NPAD, 128), jnp.float32),
        pltpu.SemaphoreType.DMA,
    ],
)
def _agg_kernel(hp_hbm, row_hbm, col_hbm, w_hbm, z2_hbm, out_hbm,
                ridx, ridx2, cidx, wbuf, gbuf, sagg, sem):
    c = lax.axis_index("c")
    s = lax.axis_index("s")
    pltpu.sync_copy(z2_hbm, sagg.at[pl.ds(s * SLC, SLC)])
    plsc.subcore_barrier()
    ept = E // NS
    base = s * ept
    off = c * N

    @pl.loop(0, ept // K)
    def _chunk(i):
        b = base + i * K
        pltpu.sync_copy(row_hbm.at[pl.ds(b, K)], ridx)
        pltpu.sync_copy(col_hbm.at[pl.ds(b, K)], cidx)
        pltpu.sync_copy(w_hbm.at[pl.ds(b, K)], wbuf)

        @pl.loop(0, K // 16)
        def _adj(j):
            ridx2[pl.ds(j * 16, 16)] = ridx[pl.ds(j * 16, 16)] + off

        pltpu.async_copy(hp_hbm.at[ridx2], gbuf, sem).wait()

        @pl.loop(0, K // 16)
        def _scale(g):
            wv16 = wbuf[pl.ds(g * 16, 16)]
            for t in range(16):
                wv = jnp.full((16,), wv16[t], jnp.float32)
                e = g * 16 + t
                for j in range(8):
                    gbuf[e, pl.ds(j * 16, 16)] = gbuf[e, pl.ds(j * 16, 16)] * wv

        pltpu.sync_copy(gbuf, sagg.at[cidx], add=True)

    plsc.subcore_barrier()
    pltpu.sync_copy(sagg.at[pl.ds(s * SLC, SLC)],
                    out_hbm.at[c, pl.ds(s * SLC, SLC)])


_R = 1000


def _tc1_body(x_ref, w1_ref, deg_ref, hp_ref, s1_ref):
    dinv = lax.rsqrt(deg_ref[:, 0] + deg_ref[:, 1] + 1.0)
    hw = jnp.dot(x_ref[...], w1_ref[...], preferred_element_type=jnp.float32)
    hp_ref[0] = dinv[:, None] * hw
    s1_ref[...] = (dinv * dinv)[:, None] * hw


def _tc2_body(agg_ref, s1_ref, deg_ref, b1_ref, w2_ref, hp2_ref, s2_ref):
    dinv = lax.rsqrt(deg_ref[:, 0] + deg_ref[:, 1] + 1.0)
    aggc = jnp.concatenate([agg_ref[0], agg_ref[1]], axis=1)
    h1 = jnp.maximum(dinv[:, None] * aggc + s1_ref[...] + b1_ref[...], 0.0)
    hw2 = jnp.dot(h1, w2_ref[...], preferred_element_type=jnp.float32)
    hp2_ref[0] = dinv[:, None] * hw2[:, :128]
    hp2_ref[1] = dinv[:, None] * hw2[:, 128:]
    s2_ref[...] = (dinv * dinv)[:, None] * hw2


def _tc3_body(agg_ref, s2_ref, deg_ref, b2_ref, batch_ref, wr_ref, br_ref,
              out_ref, acc):
    r = pl.program_id(0)

    @pl.when(r == 0)
    def _():
        acc[...] = jnp.full((B, D_H), -jnp.inf, jnp.float32)

    dinv = lax.rsqrt(deg_ref[:, 0] + deg_ref[:, 1] + 1.0)
    aggc = jnp.concatenate([agg_ref[0], agg_ref[1]], axis=1)
    h2 = dinv[:, None] * aggc + s2_ref[...] + b2_ref[...]
    bv = batch_ref[...]
    rows = [jnp.max(jnp.where(bv == g, h2, -jnp.inf), axis=0) for g in range(B)]
    acc[...] = jnp.maximum(acc[...], jnp.stack(rows))

    @pl.when(r == (N // _R) - 1)
    def _():
        pooled = acc[...]
        out_ref[...] = jnp.maximum(
            jnp.dot(pooled, wr_ref[...], preferred_element_type=jnp.float32)
            + br_ref[...], 0.0)


def kernel(x, edge_index, edge_attr, batch, W1, b1, W2, b2, Wr, br):
    f32 = jnp.float32
    row = edge_index[0]
    col = edge_index[1]
    w = edge_attr[:, 0]
    z1 = jnp.zeros((SLC,), f32)
    z2 = jnp.zeros((SLC, 128), f32)
    b1r = b1.reshape(1, D_H)
    b2r = b2.reshape(1, D_H)
    brr = br.reshape(1, D_OUT)
    batch2 = batch.astype(jnp.int32).reshape(N, 1)

    deg_pad = _deg_kernel(col, w, z1)
    deg2 = deg_pad[:, :N].T

    hp1, s1 = pl.pallas_call(
        _tc1_body,
        grid=(2, N // _R),
        in_specs=[
            pl.BlockSpec((_R, D_IN), lambda c, r: (r, 0)),
            pl.BlockSpec((D_IN, 128), lambda c, r: (0, c)),
            pl.BlockSpec((_R, 2), lambda c, r: (r, 0)),
        ],
        out_specs=[
            pl.BlockSpec((1, _R, 128), lambda c, r: (c, r, 0)),
            pl.BlockSpec((_R, 128), lambda c, r: (r, c)),
        ],
        out_shape=[
            jax.ShapeDtypeStruct((2, N, 128), f32),
            jax.ShapeDtypeStruct((N, D_H), f32),
        ],
    )(x, W1, deg2)

    agg1 = _agg_kernel(hp1.reshape(2 * N, 128), row, col, w, z2)[:, :N]

    hp2, s2 = pl.pallas_call(
        _tc2_body,
        grid=(N // _R,),
        in_specs=[
            pl.BlockSpec((2, _R, 128), lambda r: (0, r, 0)),
            pl.BlockSpec((_R, D_H), lambda r: (r, 0)),
            pl.BlockSpec((_R, 2), lambda r: (r, 0)),
            pl.BlockSpec((1, D_H), lambda r: (0, 0)),
            pl.BlockSpec((D_H, D_H), lambda r: (0, 0)),
        ],
        out_specs=[
            pl.BlockSpec((2, _R, 128), lambda r: (0, r, 0)),
            pl.BlockSpec((_R, D_H), lambda r: (r, 0)),
        ],
        out_shape=[
            jax.ShapeDtypeStruct((2, N, 128), f32),
            jax.ShapeDtypeStruct((N, D_H), f32),
        ],
    )(agg1, s1, deg2, b1r, W2)

    agg2 = _agg_kernel(hp2.reshape(2 * N, 128), row, col, w, z2)[:, :N]

    out = pl.pallas_call(
        _tc3_body,
        grid=(N // _R,),
        in_specs=[
            pl.BlockSpec((2, _R, 128), lambda r: (0, r, 0)),
            pl.BlockSpec((_R, D_H), lambda r: (r, 0)),
            pl.BlockSpec((_R, 2), lambda r: (r, 0)),
            pl.BlockSpec((1, D_H), lambda r: (0, 0)),
            pl.BlockSpec((_R, 1), lambda r: (r, 0)),
            pl.BlockSpec((D_H, D_OUT), lambda r: (0, 0)),
            pl.BlockSpec((1, D_OUT), lambda r: (0, 0)),
        ],
        out_specs=pl.BlockSpec((B, D_OUT), lambda r: (0, 0)),
        out_shape=jax.ShapeDtypeStruct((B, D_OUT), f32),
        scratch_shapes=[pltpu.VMEM((B, D_H), f32)],
    )(agg2, s2, deg2, b2r, batch2, Wr, brr)
    return out
